# baseline (speedup 1.0000x reference)
import functools
import jax
import jax.numpy as jnp
from jax import lax
from jax.experimental import pallas as pl
from jax.experimental.pallas import tpu as pltpu
from jax.experimental.pallas import tpu_sc as plsc

DIM = 64
NUM_CORES = 2
NUM_SUBCORES = 16
NUM_WORKERS = NUM_CORES * NUM_SUBCORES
CHUNK = 128
NBUF = 3


def _sc_lookup(fixed_table, trainable_table, regular_table, x_flat, v2c, v2r):
    n = x_flat.shape[0]
    per_w = n // NUM_WORKERS
    n_chunks = per_w // CHUNK
    n_groups = (n_chunks + NBUF - 1) // NBUF
    mesh = plsc.VectorSubcoreMesh(core_axis_name="c", subcore_axis_name="s")

    row_slots = [
        [pltpu.VMEM((CHUNK, DIM), jnp.float32) for _ in range(4)]
        for _ in range(NBUF)
    ]

    @functools.partial(
        pl.kernel,
        out_type=jax.ShapeDtypeStruct((n, DIM), jnp.float32),
        mesh=mesh,
        compiler_params=pltpu.CompilerParams(use_tc_tiling_on_sc=False),
        scratch_types=[
            pltpu.VMEM((per_w,), jnp.int32),
            pltpu.VMEM((per_w,), jnp.int32),
            pltpu.VMEM((per_w,), jnp.int32),
            row_slots,
            [pltpu.SemaphoreType.DMA for _ in range(NBUF)],
            [pltpu.SemaphoreType.DMA for _ in range(NBUF)],
            pltpu.SemaphoreType.DMA,
        ],
    )
    def body(fixed_h, train_h, reg_h, x_h, v2c_h, v2r_h, out_h,
             xv, cv, rv, slots, gsems, ssems, sidx):
        wid = lax.axis_index("s") * NUM_CORES + lax.axis_index("c")
        base_w = wid * per_w

        pltpu.sync_copy(x_h.at[pl.ds(base_w, per_w)], xv)
        d0 = pltpu.async_copy(v2c_h.at[xv], cv, sidx)
        d1 = pltpu.async_copy(v2r_h.at[xv], rv, sidx)
        d0.wait()
        d1.wait()

        def fire(g, b):
            sl = pl.ds(g * CHUNK, CHUNK)
            a, tb, tc, _ = slots[b]
            pltpu.async_copy(fixed_h.at[cv.at[sl]], a, gsems[b])
            pltpu.async_copy(train_h.at[cv.at[sl]], tb, gsems[b])
            pltpu.async_copy(reg_h.at[rv.at[sl]], tc, gsems[b])

        def process(g, b):
            a, tb, tc, ob = slots[b]
            pltpu.make_async_copy(fixed_h.at[pl.ds(0, CHUNK)], a, gsems[b]).wait()
            pltpu.make_async_copy(train_h.at[pl.ds(0, CHUNK)], tb, gsems[b]).wait()
            pltpu.make_async_copy(reg_h.at[pl.ds(0, CHUNK)], tc, gsems[b]).wait()

            def add_row(i, carry):
                for q in range(DIM // 16):
                    qsl = pl.ds(q * 16, 16)
                    ob[i, qsl] = a[i, qsl] + tb[i, qsl] + tc[i, qsl]
                return carry

            lax.fori_loop(0, CHUNK, add_row, 0, unroll=4)
            pltpu.async_copy(ob, out_h.at[pl.ds(base_w + g * CHUNK, CHUNK)], ssems[b])

        def drain_store(b):
            ob = slots[b][3]
            pltpu.make_async_copy(fixed_h.at[pl.ds(0, CHUNK)], ob, ssems[b]).wait()

        for b in range(NBUF - 1):
            fire(b, b)

        def group(gg, carry):
            for b in range(NBUF):
                g = gg * NBUF + b
                gf = g + NBUF - 1
                sf = (b + NBUF - 1) % NBUF

                @pl.when((gf < n_chunks) & (g >= 1))
                def _():
                    drain_store(sf)
                    fire(gf, sf)

                if b == 0:
                    @pl.when((gf < n_chunks) & (g == 0))
                    def _():
                        fire(gf, sf)

                @pl.when(g < n_chunks)
                def _():
                    process(g, b)
            return carry

        lax.fori_loop(0, n_groups, group, 0)

        for b in range(NBUF):
            drain_store(b)

    return body(fixed_table, trainable_table, regular_table, x_flat, v2c, v2r)


def kernel(fixed_table, trainable_table, regular_table, x, vocab_to_custom, vocab_to_regular):
    b, l = x.shape
    x_flat = jnp.reshape(x, (b * l,)).astype(jnp.int32)
    v2c = vocab_to_custom.astype(jnp.int32)
    v2r = vocab_to_regular.astype(jnp.int32)
    out = _sc_lookup(fixed_table, trainable_table, regular_table, x_flat, v2c, v2r)
    return jnp.reshape(out, (b, l, DIM))


# speedup vs baseline: 1.8680x; 1.8680x over previous
import functools
import jax
import jax.numpy as jnp
from jax import lax
from jax.experimental import pallas as pl
from jax.experimental.pallas import tpu as pltpu
from jax.experimental.pallas import tpu_sc as plsc

DIM = 64
HDIM = DIM // 2
NUM_CORES = 2
NUM_SUBCORES = 16
NUM_WORKERS = NUM_CORES * NUM_SUBCORES
CHUNK = 128
NBUF = 3
FT_ROWS = 20001
VOCAB = 100000
VB_PER_TILE = 6248
VB_MAIN = VB_PER_TILE * NUM_SUBCORES
VB_TAIL = VOCAB - VB_MAIN
FT_PER_TILE = 1248
FT_MAIN = FT_PER_TILE * NUM_SUBCORES
FT_TAIL = FT_ROWS - FT_MAIN


def _pack_rows(x):
    bits = lax.bitcast_convert_type(x, jnp.uint32) + jnp.uint32(0x8000)
    lo = jnp.right_shift(bits, jnp.uint32(16))
    hi = jnp.bitwise_and(bits, jnp.uint32(0xFFFF0000))
    p01 = jnp.bitwise_or(lo[:, 0:16], hi[:, 16:32])
    p23 = jnp.bitwise_or(lo[:, 32:48], hi[:, 48:64])
    return lax.bitcast_convert_type(jnp.concatenate([p01, p23], axis=1), jnp.int32)


def _tc_pack(fixed_table, trainable_table, regular_table):
    reg_rows = regular_table.shape[0]

    def ft_body(f_ref, t_ref, o_ref):
        o_ref[...] = _pack_rows(f_ref[...] + t_ref[...])

    ftpk = pl.pallas_call(
        ft_body,
        out_shape=jax.ShapeDtypeStruct((FT_ROWS, HDIM), jnp.int32),
    )(fixed_table, trainable_table)

    def reg_body(r_ref, o_ref):
        o_ref[...] = _pack_rows(r_ref[...])

    grid = 16
    blk = reg_rows // grid
    regpk = pl.pallas_call(
        reg_body,
        grid=(grid,),
        in_specs=[pl.BlockSpec((blk, DIM), lambda i: (i, 0))],
        out_specs=pl.BlockSpec((blk, HDIM), lambda i: (i, 0)),
        out_shape=jax.ShapeDtypeStruct((reg_rows, HDIM), jnp.int32),
    )(regular_table)
    return ftpk, regpk


def _sc_lookup(ftpk, regpk, x_flat, v2c, v2r):
    n = x_flat.shape[0]
    per_w = n // NUM_WORKERS
    n_chunks = per_w // CHUNK
    n_groups = (n_chunks + NBUF - 1) // NBUF
    mesh = plsc.VectorSubcoreMesh(core_axis_name="c", subcore_axis_name="s")

    row_slots = [
        [pltpu.VMEM((CHUNK, HDIM), jnp.int32),
         pltpu.VMEM((CHUNK, HDIM), jnp.int32),
         pltpu.VMEM((CHUNK, DIM), jnp.float32)]
        for _ in range(NBUF)
    ]

    @functools.partial(
        pl.kernel,
        out_type=jax.ShapeDtypeStruct((n, DIM), jnp.float32),
        mesh=mesh,
        compiler_params=pltpu.CompilerParams(use_tc_tiling_on_sc=False),
        scratch_types=[
            pltpu.VMEM((per_w,), jnp.int32),
            pltpu.VMEM((per_w,), jnp.int32),
            pltpu.VMEM((per_w,), jnp.int32),
            row_slots,
            pltpu.VMEM_SHARED((VOCAB,), jnp.int32),
            pltpu.VMEM_SHARED((VOCAB,), jnp.int32),
            pltpu.VMEM_SHARED((FT_ROWS, HDIM), jnp.int32),
            [pltpu.SemaphoreType.DMA for _ in range(NBUF)],
            [pltpu.SemaphoreType.DMA for _ in range(NBUF)],
            pltpu.SemaphoreType.DMA,
        ],
    )
    def body(ftpk_h, regpk_h, x_h, v2c_h, v2r_h, out_h,
             xv, cv, rv, slots, v2c_sh, v2r_sh, ft_sh, gsems, ssems, sidx):
        sid = lax.axis_index("s")
        wid = sid * NUM_CORES + lax.axis_index("c")
        base_w = wid * per_w

        vsl = pl.ds(sid * VB_PER_TILE, VB_PER_TILE)
        vstage = xv.at[pl.ds(0, VB_PER_TILE)]
        pltpu.sync_copy(v2c_h.at[vsl], vstage)
        pltpu.sync_copy(vstage, v2c_sh.at[vsl])
        pltpu.sync_copy(v2r_h.at[vsl], vstage)
        pltpu.sync_copy(vstage, v2r_sh.at[vsl])

        a0 = slots[0][0]
        ft_full_chunks = FT_MAIN // CHUNK
        ft_k = (ft_full_chunks + NUM_SUBCORES - 1) // NUM_SUBCORES

        def ft_stage(k, carry):
            ck = sid + k * NUM_SUBCORES

            @pl.when(ck < ft_full_chunks)
            def _():
                rsl = pl.ds(ck * CHUNK, CHUNK)
                pltpu.sync_copy(ftpk_h.at[rsl], a0)
                pltpu.sync_copy(a0, ft_sh.at[rsl])
            return carry

        lax.fori_loop(0, ft_k, ft_stage, 0)

        @pl.when(sid == 0)
        def _():
            tsl = pl.ds(VB_MAIN, VB_TAIL)
            tstage = xv.at[pl.ds(0, VB_TAIL)]
            pltpu.sync_copy(v2c_h.at[tsl], tstage)
            pltpu.sync_copy(tstage, v2c_sh.at[tsl])
            pltpu.sync_copy(v2r_h.at[tsl], tstage)
            pltpu.sync_copy(tstage, v2r_sh.at[tsl])
            ftsl = pl.ds(FT_MAIN, FT_TAIL)
            fstage = a0.at[pl.ds(0, FT_TAIL)]
            pltpu.sync_copy(ftpk_h.at[ftsl], fstage)
            pltpu.sync_copy(fstage, ft_sh.at[ftsl])

        plsc.subcore_barrier()

        pltpu.sync_copy(x_h.at[pl.ds(base_w, per_w)], xv)
        d0 = pltpu.async_copy(v2c_h.at[xv], cv, sidx)
        d1 = pltpu.async_copy(v2r_h.at[xv], rv, sidx)
        d0.wait()
        d1.wait()

        def fire(g, b):
            sl = pl.ds(g * CHUNK, CHUNK)
            a, tc, _ = slots[b]
            pltpu.async_copy(ftpk_h.at[cv.at[sl]], a, gsems[b])
            pltpu.async_copy(regpk_h.at[rv.at[sl]], tc, gsems[b])

        def process(g, b):
            a, tc, ob = slots[b]
            pltpu.make_async_copy(regpk_h.at[pl.ds(0, CHUNK)], a, gsems[b]).wait()
            pltpu.make_async_copy(regpk_h.at[pl.ds(0, CHUNK)], tc, gsems[b]).wait()

            shift16 = jnp.full((16,), 16, jnp.int32)
            maskhi = jnp.full((16,), -65536, jnp.int32)

            def add_row(i, carry):
                for h in range(2):
                    hsl = pl.ds(h * 16, 16)
                    vf = a[i, hsl]
                    vr = tc[i, hsl]
                    lo = (lax.bitcast_convert_type(lax.shift_left(vf, shift16), jnp.float32)
                          + lax.bitcast_convert_type(lax.shift_left(vr, shift16), jnp.float32))
                    hi = (lax.bitcast_convert_type(lax.bitwise_and(vf, maskhi), jnp.float32)
                          + lax.bitcast_convert_type(lax.bitwise_and(vr, maskhi), jnp.float32))
                    ob[i, pl.ds(h * 32, 16)] = lo
                    ob[i, pl.ds(h * 32 + 16, 16)] = hi
                return carry

            lax.fori_loop(0, CHUNK, add_row, 0, unroll=4)
            pltpu.async_copy(ob, out_h.at[pl.ds(base_w + g * CHUNK, CHUNK)], ssems[b])

        def drain_store(b):
            ob = slots[b][2]
            pltpu.make_async_copy(out_h.at[pl.ds(0, CHUNK)], ob, ssems[b]).wait()

        for b in range(NBUF - 1):
            fire(b, b)

        def group(gg, carry):
            for b in range(NBUF):
                g = gg * NBUF + b
                gf = g + NBUF - 1
                sf = (b + NBUF - 1) % NBUF

                @pl.when((gf < n_chunks) & (g >= 1))
                def _():
                    drain_store(sf)
                    fire(gf, sf)

                if b == 0:
                    @pl.when((gf < n_chunks) & (g == 0))
                    def _():
                        fire(gf, sf)

                @pl.when(g < n_chunks)
                def _():
                    process(g, b)
            return carry

        lax.fori_loop(0, n_groups, group, 0)

        for b in range(NBUF):
            drain_store(b)

    return body(ftpk, regpk, x_flat, v2c, v2r)


def kernel(fixed_table, trainable_table, regular_table, x, vocab_to_custom, vocab_to_regular):
    b, l = x.shape
    x_flat = jnp.reshape(x, (b * l,)).astype(jnp.int32)
    v2c = vocab_to_custom.astype(jnp.int32)
    v2r = vocab_to_regular.astype(jnp.int32)
    ftpk, regpk = _tc_pack(fixed_table, trainable_table, regular_table)
    out = _sc_lookup(ftpk, regpk, x_flat, v2c, v2r)
    return jnp.reshape(out, (b, l, DIM))


# speedup vs baseline: 1.8918x; 1.0128x over previous
import functools
import jax
import jax.numpy as jnp
from jax import lax
from jax.experimental import pallas as pl
from jax.experimental.pallas import tpu as pltpu
from jax.experimental.pallas import tpu_sc as plsc

DIM = 64
HDIM = DIM // 2
NUM_CORES = 2
NUM_SUBCORES = 16
NUM_WORKERS = NUM_CORES * NUM_SUBCORES
CHUNK = 128
NBUF = 3
FT_ROWS = 20001
VOCAB = 100000
VB_PER_TILE = 6248
VB_MAIN = VB_PER_TILE * NUM_SUBCORES
VB_TAIL = VOCAB - VB_MAIN
FT_PER_TILE = 1248
FT_MAIN = FT_PER_TILE * NUM_SUBCORES
FT_TAIL = FT_ROWS - FT_MAIN


def _pack_rows(x):
    bits = lax.bitcast_convert_type(x, jnp.uint32) + jnp.uint32(0x8000)
    lo = jnp.right_shift(bits, jnp.uint32(16))
    hi = jnp.bitwise_and(bits, jnp.uint32(0xFFFF0000))
    p01 = jnp.bitwise_or(lo[:, 0:16], hi[:, 16:32])
    p23 = jnp.bitwise_or(lo[:, 32:48], hi[:, 48:64])
    return lax.bitcast_convert_type(jnp.concatenate([p01, p23], axis=1), jnp.int32)


def _tc_pack(fixed_table, trainable_table, regular_table):
    reg_rows = regular_table.shape[0]

    def ft_body(f_ref, t_ref, o_ref):
        o_ref[...] = _pack_rows(f_ref[...] + t_ref[...])

    ftpk = pl.pallas_call(
        ft_body,
        out_shape=jax.ShapeDtypeStruct((FT_ROWS, HDIM), jnp.int32),
    )(fixed_table, trainable_table)

    def reg_body(r_ref, o_ref):
        o_ref[...] = _pack_rows(r_ref[...])

    grid = 16
    blk = reg_rows // grid
    regpk = pl.pallas_call(
        reg_body,
        grid=(grid,),
        in_specs=[pl.BlockSpec((blk, DIM), lambda i: (i, 0))],
        out_specs=pl.BlockSpec((blk, HDIM), lambda i: (i, 0)),
        out_shape=jax.ShapeDtypeStruct((reg_rows, HDIM), jnp.int32),
    )(regular_table)
    return ftpk, regpk


def _sc_lookup(ftpk, regpk, x_flat, v2c, v2r):
    n = x_flat.shape[0]
    per_w = n // NUM_WORKERS
    n_chunks = per_w // CHUNK
    n_groups = (n_chunks + NBUF - 1) // NBUF
    mesh = plsc.VectorSubcoreMesh(core_axis_name="c", subcore_axis_name="s")

    row_slots = [
        [pltpu.VMEM((CHUNK, HDIM), jnp.int32),
         pltpu.VMEM((CHUNK, HDIM), jnp.int32),
         pltpu.VMEM((CHUNK, DIM), jnp.float32)]
        for _ in range(NBUF)
    ]

    @functools.partial(
        pl.kernel,
        out_type=jax.ShapeDtypeStruct((n, DIM), jnp.float32),
        mesh=mesh,
        compiler_params=pltpu.CompilerParams(use_tc_tiling_on_sc=False),
        scratch_types=[
            pltpu.VMEM((per_w,), jnp.int32),
            pltpu.VMEM((per_w,), jnp.int32),
            pltpu.VMEM((per_w,), jnp.int32),
            row_slots,
            pltpu.VMEM_SHARED((VOCAB,), jnp.int32),
            pltpu.VMEM_SHARED((VOCAB,), jnp.int32),
            [pltpu.SemaphoreType.DMA for _ in range(NBUF)],
            [pltpu.SemaphoreType.DMA for _ in range(NBUF)],
            pltpu.SemaphoreType.DMA,
        ],
    )
    def body(ftpk_h, regpk_h, x_h, v2c_h, v2r_h, out_h,
             xv, cv, rv, slots, v2c_sh, v2r_sh, gsems, ssems, sidx):
        sid = lax.axis_index("s")
        wid = sid * NUM_CORES + lax.axis_index("c")
        base_w = wid * per_w

        vsl = pl.ds(sid * VB_PER_TILE, VB_PER_TILE)
        vstage = xv.at[pl.ds(0, VB_PER_TILE)]
        pltpu.sync_copy(v2c_h.at[vsl], vstage)
        pltpu.sync_copy(vstage, v2c_sh.at[vsl])
        pltpu.sync_copy(v2r_h.at[vsl], vstage)
        pltpu.sync_copy(vstage, v2r_sh.at[vsl])

        @pl.when(sid == 0)
        def _():
            tsl = pl.ds(VB_MAIN, VB_TAIL)
            tstage = xv.at[pl.ds(0, VB_TAIL)]
            pltpu.sync_copy(v2c_h.at[tsl], tstage)
            pltpu.sync_copy(tstage, v2c_sh.at[tsl])
            pltpu.sync_copy(v2r_h.at[tsl], tstage)
            pltpu.sync_copy(tstage, v2r_sh.at[tsl])

        plsc.subcore_barrier()

        pltpu.sync_copy(x_h.at[pl.ds(base_w, per_w)], xv)
        d0 = pltpu.async_copy(v2c_sh.at[xv], cv, sidx)
        d1 = pltpu.async_copy(v2r_sh.at[xv], rv, sidx)
        d0.wait()
        d1.wait()

        def fire(g, b):
            sl = pl.ds(g * CHUNK, CHUNK)
            a, tc, _ = slots[b]
            pltpu.async_copy(ftpk_h.at[cv.at[sl]], a, gsems[b])
            pltpu.async_copy(regpk_h.at[rv.at[sl]], tc, gsems[b])

        def process(g, b):
            a, tc, ob = slots[b]
            pltpu.make_async_copy(regpk_h.at[pl.ds(0, CHUNK)], a, gsems[b]).wait()
            pltpu.make_async_copy(regpk_h.at[pl.ds(0, CHUNK)], tc, gsems[b]).wait()

            shift16 = jnp.full((16,), 16, jnp.int32)
            maskhi = jnp.full((16,), -65536, jnp.int32)

            def add_row(i, carry):
                for h in range(2):
                    hsl = pl.ds(h * 16, 16)
                    vf = a[i, hsl]
                    vr = tc[i, hsl]
                    lo = (lax.bitcast_convert_type(lax.shift_left(vf, shift16), jnp.float32)
                          + lax.bitcast_convert_type(lax.shift_left(vr, shift16), jnp.float32))
                    hi = (lax.bitcast_convert_type(lax.bitwise_and(vf, maskhi), jnp.float32)
                          + lax.bitcast_convert_type(lax.bitwise_and(vr, maskhi), jnp.float32))
                    ob[i, pl.ds(h * 32, 16)] = lo
                    ob[i, pl.ds(h * 32 + 16, 16)] = hi
                return carry

            lax.fori_loop(0, CHUNK, add_row, 0, unroll=4)
            pltpu.async_copy(ob, out_h.at[pl.ds(base_w + g * CHUNK, CHUNK)], ssems[b])

        def drain_store(b):
            ob = slots[b][2]
            pltpu.make_async_copy(out_h.at[pl.ds(0, CHUNK)], ob, ssems[b]).wait()

        for b in range(NBUF - 1):
            fire(b, b)

        def group(gg, carry):
            for b in range(NBUF):
                g = gg * NBUF + b
                gf = g + NBUF - 1
                sf = (b + NBUF - 1) % NBUF

                @pl.when((gf < n_chunks) & (g >= 1))
                def _():
                    drain_store(sf)
                    fire(gf, sf)

                if b == 0:
                    @pl.when((gf < n_chunks) & (g == 0))
                    def _():
                        fire(gf, sf)

                @pl.when(g < n_chunks)
                def _():
                    process(g, b)
            return carry

        lax.fori_loop(0, n_groups, group, 0)

        for b in range(NBUF):
            drain_store(b)

    return body(ftpk, regpk, x_flat, v2c, v2r)


def kernel(fixed_table, trainable_table, regular_table, x, vocab_to_custom, vocab_to_regular):
    b, l = x.shape
    x_flat = jnp.reshape(x, (b * l,)).astype(jnp.int32)
    v2c = vocab_to_custom.astype(jnp.int32)
    v2r = vocab_to_regular.astype(jnp.int32)
    ftpk, regpk = _tc_pack(fixed_table, trainable_table, regular_table)
    out = _sc_lookup(ftpk, regpk, x_flat, v2c, v2r)
    return jnp.reshape(out, (b, l, DIM))


# speedup vs baseline: 9.3077x; 4.9199x over previous
import functools
import jax
import jax.numpy as jnp
from jax import lax
from jax.experimental import pallas as pl
from jax.experimental.pallas import tpu as pltpu
from jax.experimental.pallas import tpu_sc as plsc

DIM = 64
HDIM = DIM // 2
NUM_CORES = 2
NUM_SUBCORES = 16
NUM_WORKERS = NUM_CORES * NUM_SUBCORES
CHUNK = 128
NBUF = 3
FT_ROWS = 20001
VOCAB = 100000
VB_PER_TILE = 6248
VB_MAIN = VB_PER_TILE * NUM_SUBCORES
VB_TAIL = VOCAB - VB_MAIN


def _pack_rows(x):
    bits = lax.bitcast_convert_type(x, jnp.uint32) + jnp.uint32(0x8000)
    lo = jnp.right_shift(bits, jnp.uint32(16))
    hi = jnp.bitwise_and(bits, jnp.uint32(0xFFFF0000))
    p01 = jnp.bitwise_or(lo[:, 0:16], hi[:, 16:32])
    p23 = jnp.bitwise_or(lo[:, 32:48], hi[:, 48:64])
    return lax.bitcast_convert_type(jnp.concatenate([p01, p23], axis=1), jnp.int32)


def _tc_prep(fixed_table, trainable_table, regular_table, v2c, v2r):
    reg_rows = regular_table.shape[0]

    def ft_body(f_ref, t_ref, o_ref):
        o_ref[...] = _pack_rows(f_ref[...] + t_ref[...])

    ftpk = pl.pallas_call(
        ft_body,
        out_shape=jax.ShapeDtypeStruct((FT_ROWS, HDIM), jnp.int32),
    )(fixed_table, trainable_table)

    def reg_body(r_ref, o_ref):
        o_ref[...] = _pack_rows(r_ref[...])

    grid = 16
    blk = reg_rows // grid
    regpk = pl.pallas_call(
        reg_body,
        grid=(grid,),
        in_specs=[pl.BlockSpec((blk, DIM), lambda i: (i, 0))],
        out_specs=pl.BlockSpec((blk, HDIM), lambda i: (i, 0)),
        out_shape=jax.ShapeDtypeStruct((reg_rows, HDIM), jnp.int32),
    )(regular_table)

    def remap_body(c_ref, r_ref, o_ref):
        c = c_ref[...]
        o_ref[...] = jnp.where(c > 0, c, r_ref[...] + FT_ROWS)

    v2u = pl.pallas_call(
        remap_body,
        out_shape=jax.ShapeDtypeStruct((100, VOCAB // 100), jnp.int32),
    )(jnp.reshape(v2c, (100, VOCAB // 100)), jnp.reshape(v2r, (100, VOCAB // 100)))

    return jnp.concatenate([ftpk, regpk], axis=0), jnp.reshape(v2u, (VOCAB,))


def _sc_lookup(upk, v2u, x_flat):
    n = x_flat.shape[0]
    per_w = n // NUM_WORKERS
    n_chunks = per_w // CHUNK
    n_groups = (n_chunks + NBUF - 1) // NBUF
    mesh = plsc.VectorSubcoreMesh(core_axis_name="c", subcore_axis_name="s")

    slots_spec = [
        [pltpu.VMEM((CHUNK,), jnp.int32),
         pltpu.VMEM((CHUNK, HDIM), jnp.int32),
         pltpu.VMEM((CHUNK, DIM), jnp.float32)]
        for _ in range(NBUF)
    ]

    @functools.partial(
        pl.kernel,
        out_type=jax.ShapeDtypeStruct((n, DIM), jnp.float32),
        mesh=mesh,
        compiler_params=pltpu.CompilerParams(use_tc_tiling_on_sc=False),
        scratch_types=[
            pltpu.VMEM((per_w,), jnp.int32),
            slots_spec,
            pltpu.VMEM_SHARED((VOCAB,), jnp.int32),
            [pltpu.SemaphoreType.DMA for _ in range(NBUF)],
            [pltpu.SemaphoreType.DMA for _ in range(NBUF)],
            [pltpu.SemaphoreType.DMA for _ in range(NBUF)],
        ],
    )
    def body(upk_h, v2u_h, x_h, out_h, xv, slots, v2u_sh, isems, gsems, ssems):
        sid = lax.axis_index("s")
        wid = sid * NUM_CORES + lax.axis_index("c")
        base_w = wid * per_w

        vsl = pl.ds(sid * VB_PER_TILE, VB_PER_TILE)
        vstage = xv.at[pl.ds(0, VB_PER_TILE)]
        pltpu.sync_copy(v2u_h.at[vsl], vstage)
        pltpu.sync_copy(vstage, v2u_sh.at[vsl])

        @pl.when(sid == 0)
        def _():
            tsl = pl.ds(VB_MAIN, VB_TAIL)
            tstage = xv.at[pl.ds(0, VB_TAIL)]
            pltpu.sync_copy(v2u_h.at[tsl], tstage)
            pltpu.sync_copy(tstage, v2u_sh.at[tsl])

        plsc.subcore_barrier()

        pltpu.sync_copy(x_h.at[pl.ds(base_w, per_w)], xv)

        def fire_idx(g, b):
            uidx = slots[b][0]
            xsl = xv.at[pl.ds(g * CHUNK, CHUNK)]
            pltpu.async_copy(v2u_sh.at[xsl], uidx, isems[b])

        def fire_rows(g, b):
            uidx, a, _ = slots[b]
            pltpu.make_async_copy(v2u_h.at[pl.ds(0, CHUNK)], uidx, isems[b]).wait()
            pltpu.async_copy(upk_h.at[uidx], a, gsems[b])

        shift16 = jnp.full((16,), 16, jnp.int32)
        maskhi = jnp.full((16,), -65536, jnp.int32)

        def process(g, b):
            _, a, ob = slots[b]
            pltpu.make_async_copy(upk_h.at[pl.ds(0, CHUNK)], a, gsems[b]).wait()

            @pl.when(g >= NBUF)
            def _():
                pltpu.make_async_copy(out_h.at[pl.ds(0, CHUNK)], ob, ssems[b]).wait()

            def unpack_row(i, carry):
                for h in range(2):
                    v = a[i, pl.ds(h * 16, 16)]
                    ob[i, pl.ds(h * 32, 16)] = lax.bitcast_convert_type(
                        lax.shift_left(v, shift16), jnp.float32)
                    ob[i, pl.ds(h * 32 + 16, 16)] = lax.bitcast_convert_type(
                        lax.bitwise_and(v, maskhi), jnp.float32)
                return carry

            lax.fori_loop(0, CHUNK, unpack_row, 0, unroll=4)
            pltpu.async_copy(ob, out_h.at[pl.ds(base_w + g * CHUNK, CHUNK)], ssems[b])

        fire_idx(0, 0)
        fire_idx(1, 1)
        fire_rows(0, 0)

        def group(gg, carry):
            for b in range(NBUF):
                g = gg * NBUF + b

                @pl.when(g + 2 < n_chunks)
                def _():
                    fire_idx(g + 2, (b + 2) % NBUF)

                @pl.when(g + 1 < n_chunks)
                def _():
                    fire_rows(g + 1, (b + 1) % NBUF)

                @pl.when(g < n_chunks)
                def _():
                    process(g, b)
            return carry

        lax.fori_loop(0, n_groups, group, 0)

        for b in range(NBUF):
            ob = slots[b][2]
            pltpu.make_async_copy(out_h.at[pl.ds(0, CHUNK)], ob, ssems[b]).wait()

    return body(upk, v2u, x_flat)


def kernel(fixed_table, trainable_table, regular_table, x, vocab_to_custom, vocab_to_regular):
    b, l = x.shape
    x_flat = jnp.reshape(x, (b * l,)).astype(jnp.int32)
    v2c = vocab_to_custom.astype(jnp.int32)
    v2r = vocab_to_regular.astype(jnp.int32)
    upk, v2u = _tc_prep(fixed_table, trainable_table, regular_table, v2c, v2r)
    out = _sc_lookup(upk, v2u, x_flat)
    return jnp.reshape(out, (b, l, DIM))
